# baseline (speedup 1.0000x reference)
import functools

import jax
import jax.numpy as jnp
from jax import lax
from jax.experimental import pallas as pl

_BT = 256


def _body(E, H, x_ref, gwt_ref, w1_ref, b1_ref, w2_ref, b2_ref,
          ws1_ref, bs1_ref, ws2_ref, bs2_ref, out_ref):
    x = x_ref[...]
    l = jnp.dot(x, gwt_ref[...], preferred_element_type=jnp.float32)
    mx = jnp.max(l, axis=1, keepdims=True)
    p = jnp.exp(l - mx)
    z = jnp.sum(p, axis=1, keepdims=True)
    i8 = lax.broadcasted_iota(jnp.int32, (_BT, E), 1)
    is1 = l == mx
    idx1 = jnp.min(jnp.where(is1, i8, E), axis=1, keepdims=True)
    lm = jnp.where(i8 == idx1, -jnp.inf, l)
    mx2 = jnp.max(lm, axis=1, keepdims=True)
    idx2 = jnp.min(jnp.where(lm == mx2, i8, E), axis=1, keepdims=True)
    p1 = 1.0 / z
    p2 = jnp.exp(mx2 - mx) / z
    den = p1 + p2 + 1e-20
    m = (jnp.where(i8 == idx1, p1 / den, 0.0)
         + jnp.where(i8 == idx2, p2 / den, 0.0))
    xb = x.astype(jnp.bfloat16)
    h = jnp.maximum(
        jnp.dot(xb, w1_ref[...], preferred_element_type=jnp.float32)
        + b1_ref[...], 0.0)
    expand = (lax.broadcasted_iota(jnp.int32, (E, E * H), 1) // H
              == lax.broadcasted_iota(jnp.int32, (E, E * H), 0)
              ).astype(jnp.float32)
    gate = jnp.dot(m, expand, preferred_element_type=jnp.float32)
    hw = (h * gate).astype(jnp.bfloat16)
    y = jnp.dot(hw, w2_ref[...], preferred_element_type=jnp.float32)
    y = y + jnp.dot(m, b2_ref[...], preferred_element_type=jnp.float32)
    s = jnp.maximum(
        jnp.dot(xb, ws1_ref[...], preferred_element_type=jnp.float32)
        + bs1_ref[...], 0.0)
    s = jnp.dot(s.astype(jnp.bfloat16), ws2_ref[...],
                preferred_element_type=jnp.float32) + bs2_ref[...]
    out_ref[...] = y + s


def _prep_body(w1_ref, w2_ref, ws1_ref, ws2_ref,
               w1f_ref, w2f_ref, ws1f_ref, ws2f_ref):
    w1f_ref[...] = w1_ref[0].astype(jnp.bfloat16)
    w2f_ref[...] = w2_ref[...].astype(jnp.bfloat16)
    ws1f_ref[...] = ws1_ref[...].astype(jnp.bfloat16)
    ws2f_ref[...] = ws2_ref[...].astype(jnp.bfloat16)


def kernel(hidden_states, gate_w, w1, b1, w2, b2, ws1, bs1, ws2, bs2):
    b, s, d = hidden_states.shape
    T = b * s
    E, D, H = w1.shape
    EH = E * H
    HS = ws1.shape[1]
    x = hidden_states.reshape(T, d)

    w1f, w2f, ws1f, ws2f = pl.pallas_call(
        _prep_body,
        grid=(E,),
        in_specs=[
            pl.BlockSpec((1, D, H), lambda e: (e, 0, 0)),
            pl.BlockSpec((EH // E, D), lambda e: (e, 0)),
            pl.BlockSpec((D // E, HS), lambda e: (e, 0)),
            pl.BlockSpec((HS // E, D), lambda e: (e, 0)),
        ],
        out_specs=(
            pl.BlockSpec((D, H), lambda e: (0, e)),
            pl.BlockSpec((EH // E, D), lambda e: (e, 0)),
            pl.BlockSpec((D // E, HS), lambda e: (e, 0)),
            pl.BlockSpec((HS // E, D), lambda e: (e, 0)),
        ),
        out_shape=(
            jax.ShapeDtypeStruct((D, EH), jnp.bfloat16),
            jax.ShapeDtypeStruct((EH, D), jnp.bfloat16),
            jax.ShapeDtypeStruct((D, HS), jnp.bfloat16),
            jax.ShapeDtypeStruct((HS, D), jnp.bfloat16),
        ),
    )(w1, w2.reshape(EH, D), ws1, ws2)
    b1f = b1.reshape(1, EH)
    out = pl.pallas_call(
        functools.partial(_body, E, H),
        grid=(T // _BT,),
        in_specs=[
            pl.BlockSpec((_BT, D), lambda i: (i, 0)),
            pl.BlockSpec((D, E), lambda i: (0, 0)),
            pl.BlockSpec((D, EH), lambda i: (0, 0)),
            pl.BlockSpec((1, EH), lambda i: (0, 0)),
            pl.BlockSpec((EH, D), lambda i: (0, 0)),
            pl.BlockSpec((E, D), lambda i: (0, 0)),
            pl.BlockSpec((D, HS), lambda i: (0, 0)),
            pl.BlockSpec((1, HS), lambda i: (0, 0)),
            pl.BlockSpec((HS, D), lambda i: (0, 0)),
            pl.BlockSpec((1, D), lambda i: (0, 0)),
        ],
        out_specs=pl.BlockSpec((_BT, D), lambda i: (i, 0)),
        out_shape=jax.ShapeDtypeStruct((T, D), jnp.float32),
    )(x, gate_w.T, w1f, b1f, w2f, b2,
      ws1f, bs1.reshape(1, HS),
      ws2f, bs2.reshape(1, D))

    return out.reshape(b, s, d)


# speedup vs baseline: 1.0197x; 1.0197x over previous
import functools

import jax
import jax.numpy as jnp
from jax import lax
from jax.experimental import pallas as pl

_BT = 256


def _body(E, H, x_ref, gwt_ref, w1_ref, b1_ref, w2_ref, b2_ref,
          ws1_ref, bs1_ref, ws2_ref, bs2_ref, out_ref):
    x = x_ref[...]
    l = jnp.dot(x, gwt_ref[...], preferred_element_type=jnp.float32)
    mx = jnp.max(l, axis=1, keepdims=True)
    p = jnp.exp(l - mx)
    z = jnp.sum(p, axis=1, keepdims=True)
    i8 = lax.broadcasted_iota(jnp.int32, (_BT, E), 1)
    is1 = l == mx
    idx1 = jnp.min(jnp.where(is1, i8, E), axis=1, keepdims=True)
    lm = jnp.where(i8 == idx1, -jnp.inf, l)
    mx2 = jnp.max(lm, axis=1, keepdims=True)
    idx2 = jnp.min(jnp.where(lm == mx2, i8, E), axis=1, keepdims=True)
    p1 = 1.0 / z
    p2 = jnp.exp(mx2 - mx) / z
    den = p1 + p2 + 1e-20
    m = (jnp.where(i8 == idx1, p1 / den, 0.0)
         + jnp.where(i8 == idx2, p2 / den, 0.0))
    xb = x.astype(jnp.bfloat16)
    h = jnp.maximum(
        jnp.dot(xb, w1_ref[...], preferred_element_type=jnp.float32)
        + b1_ref[...], 0.0)
    hw = jnp.concatenate(
        [h[:, e * H:(e + 1) * H] * m[:, e:e + 1] for e in range(E)],
        axis=1).astype(jnp.bfloat16)
    y = jnp.dot(hw, w2_ref[...], preferred_element_type=jnp.float32)
    y = y + jnp.dot(m, b2_ref[...], preferred_element_type=jnp.float32)
    s = jnp.maximum(
        jnp.dot(xb, ws1_ref[...], preferred_element_type=jnp.float32)
        + bs1_ref[...], 0.0)
    s = jnp.dot(s.astype(jnp.bfloat16), ws2_ref[...],
                preferred_element_type=jnp.float32) + bs2_ref[...]
    out_ref[...] = y + s


def kernel(hidden_states, gate_w, w1, b1, w2, b2, ws1, bs1, ws2, bs2):
    b, s, d = hidden_states.shape
    T = b * s
    E, D, H = w1.shape
    EH = E * H
    HS = ws1.shape[1]
    x = hidden_states.reshape(T, d)

    w1f = w1.transpose(1, 0, 2).reshape(D, EH).astype(jnp.bfloat16)
    b1f = b1.reshape(1, EH)
    out = pl.pallas_call(
        functools.partial(_body, E, H),
        grid=(T // _BT,),
        in_specs=[
            pl.BlockSpec((_BT, D), lambda i: (i, 0)),
            pl.BlockSpec((D, E), lambda i: (0, 0)),
            pl.BlockSpec((D, EH), lambda i: (0, 0)),
            pl.BlockSpec((1, EH), lambda i: (0, 0)),
            pl.BlockSpec((EH, D), lambda i: (0, 0)),
            pl.BlockSpec((E, D), lambda i: (0, 0)),
            pl.BlockSpec((D, HS), lambda i: (0, 0)),
            pl.BlockSpec((1, HS), lambda i: (0, 0)),
            pl.BlockSpec((HS, D), lambda i: (0, 0)),
            pl.BlockSpec((1, D), lambda i: (0, 0)),
        ],
        out_specs=pl.BlockSpec((_BT, D), lambda i: (i, 0)),
        out_shape=jax.ShapeDtypeStruct((T, D), jnp.float32),
    )(x, gate_w.T, w1f, b1f, w2.reshape(EH, D).astype(jnp.bfloat16), b2,
      ws1.astype(jnp.bfloat16), bs1.reshape(1, HS),
      ws2.astype(jnp.bfloat16), bs2.reshape(1, D))

    return out.reshape(b, s, d)
